# Initial kernel scaffold; baseline (speedup 1.0000x reference)
import jax
import jax.numpy as jnp
from jax.experimental import pallas as pl


def kernel(x, W1, g1, b1, W2, g2, b2, Wout, gout, bout):
    raise NotImplementedError("write your pallas kernel here")


# speedup vs baseline: 6.3247x; 6.3247x over previous
import jax
import jax.numpy as jnp
from jax import lax
from jax.experimental import pallas as pl

K = 20
TILE = 256
F32 = jnp.float32


def _dot(a, b, dims):
    return lax.dot_general(a, b, (dims, ((), ())), preferred_element_type=F32)


def _p0_kernel(x_ref, w1a_ref, wz_ref, y_ref, z_ref):
    xb = x_ref[0]
    y_ref[0] = _dot(xb, w1a_ref[...], ((0,), (1,)))
    z_ref[0] = _dot(xb, wz_ref[...], ((0,), (1,)))


def _p1_kernel(xt_ref, xb_ref, yb_ref, z_ref,
               g_ref, idx_ref, sumh_ref, sumsq_ref):
    b = pl.program_id(0)
    t = pl.program_id(1)
    n_total = xb_ref.shape[2]
    xt = xt_ref[0]
    xb = xb_ref[0]
    yb = yb_ref[0]
    z = z_ref[0]

    inner = _dot(xt, xb, ((0,), (0,)))
    xxp = jnp.sum(xt * xt, axis=0)
    xxb = jnp.sum(xb * xb, axis=0)
    d = 2.0 * inner - xxp[:, None] - xxb[None, :]

    iota = lax.broadcasted_iota(jnp.int32, d.shape, 1)
    neg = jnp.float32(-jnp.inf)
    sh = jnp.zeros((z.shape[1],), F32)
    sq = jnp.zeros((z.shape[1],), F32)
    for j in range(K):
        vmax = jnp.max(d, axis=1, keepdims=True)
        cand = jnp.where(d == vmax, iota, n_total)
        m = jnp.min(cand, axis=1, keepdims=True)
        hit = iota == m
        d = jnp.where(hit, neg, d)
        gj = _dot(hit.astype(F32), yb, ((1,), (0,)))
        g_ref[0, :, j, :] = gj
        idx_ref[0, :, pl.ds(j, 1)] = m + b * n_total
        hj = gj + z
        sh = sh + jnp.sum(hj, axis=0)
        sq = sq + jnp.sum(hj * hj, axis=0)

    @pl.when(jnp.logical_and(b == 0, t == 0))
    def _init():
        sumh_ref[...] = jnp.zeros_like(sumh_ref)
        sumsq_ref[...] = jnp.zeros_like(sumsq_ref)
    sumh_ref[0, :] = sumh_ref[0, :] + sh
    sumsq_ref[0, :] = sumsq_ref[0, :] + sq


def _p2_kernel(g_ref, z_ref, s1_ref, t1_ref,
               x1_ref, muhr_ref, mhr_ref):
    b = pl.program_id(0)
    t = pl.program_id(1)
    z = z_ref[0]
    s1 = s1_ref[...]
    t1 = t1_ref[...]
    hrs = []
    for j in range(K):
        h = g_ref[0, :, j, :] + z
        hrs.append(jnp.maximum(h * s1 + t1, 0.0))
    mx = hrs[0]
    for j in range(1, K):
        mx = jnp.maximum(mx, hrs[j])
    ssum = jnp.zeros_like(mx)
    num = jnp.zeros_like(mx)
    for j in range(K):
        e = jnp.exp(hrs[j] - mx)
        ssum = ssum + e
        num = num + hrs[j] * e
    x1_ref[0] = num / ssum

    mu = jnp.zeros((z.shape[1],), F32)
    mm = jnp.zeros((z.shape[1], z.shape[1]), F32)
    for j in range(K):
        mu = mu + jnp.sum(hrs[j], axis=0)
        mm = mm + _dot(hrs[j], hrs[j], ((0,), (0,)))

    @pl.when(jnp.logical_and(b == 0, t == 0))
    def _init():
        muhr_ref[...] = jnp.zeros_like(muhr_ref)
        mhr_ref[...] = jnp.zeros_like(mhr_ref)
    muhr_ref[0, :] = muhr_ref[0, :] + mu
    mhr_ref[...] = mhr_ref[...] + mm


def _p2b_kernel(mu_ref, mm_ref, w2_ref, g2_ref, b2_ref, n_samples,
                s2_ref, t2_ref):
    minv = jnp.float32(1.0 / n_samples)
    w2 = w2_ref[...]
    mean_hr = mu_ref[...] * minv
    mean_c = _dot(w2, mean_hr, ((1,), (1,)))
    u = _dot(w2, mm_ref[...] * minv, ((1,), (0,)))
    e2 = jnp.sum(u * w2, axis=1, keepdims=True)
    var = e2 - mean_c * mean_c
    rstd = lax.rsqrt(var + 1e-5)
    g2 = g2_ref[...]
    sc = g2 * rstd
    tc = b2_ref[...] - mean_c * sc
    s2_ref[...] = jnp.reshape(sc, s2_ref.shape)
    t2_ref[...] = jnp.reshape(tc, t2_ref.shape)


def _p3_kernel(g_ref, z_ref, s1_ref, t1_ref, s2_ref, t2_ref, w2_ref, x1_ref,
               x2_ref, muv_ref, mv_ref):
    b = pl.program_id(0)
    t = pl.program_id(1)
    z = z_ref[0]
    s1 = s1_ref[...]
    t1 = t1_ref[...]
    s2 = s2_ref[...]
    t2 = t2_ref[...]
    w2 = w2_ref[...]
    hr2s = []
    for j in range(K):
        h = g_ref[0, :, j, :] + z
        hr = jnp.maximum(h * s1 + t1, 0.0)
        h2 = _dot(hr, w2, ((1,), (1,)))
        hr2s.append(jnp.maximum(h2 * s2 + t2, 0.0))
    mx = hr2s[0]
    for j in range(1, K):
        mx = jnp.maximum(mx, hr2s[j])
    ssum = jnp.zeros_like(mx)
    num = jnp.zeros_like(mx)
    for j in range(K):
        e = jnp.exp(hr2s[j] - mx)
        ssum = ssum + e
        num = num + hr2s[j] * e
    x2 = num / ssum
    x2_ref[0] = x2

    v = jnp.concatenate([x1_ref[0], x2], axis=1)
    mu = jnp.sum(v, axis=0)
    mm = _dot(v, v, ((0,), (0,)))

    @pl.when(jnp.logical_and(b == 0, t == 0))
    def _init():
        muv_ref[...] = jnp.zeros_like(muv_ref)
        mv_ref[...] = jnp.zeros_like(mv_ref)
    muv_ref[0, :] = muv_ref[0, :] + mu
    mv_ref[...] = mv_ref[...] + mm


def _p3b_kernel(mu_ref, mm_ref, wo_ref, go_ref, bo_ref, n_samples,
                so_ref, to_ref):
    minv = jnp.float32(1.0 / n_samples)
    wo = wo_ref[...]
    mean_v = mu_ref[...] * minv
    mean_c = _dot(wo, mean_v, ((1,), (1,)))
    u = _dot(wo, mm_ref[...] * minv, ((1,), (0,)))
    e2 = jnp.sum(u * wo, axis=1, keepdims=True)
    var = e2 - mean_c * mean_c
    rstd = lax.rsqrt(var + 1e-5)
    go = go_ref[...]
    so_ref[...] = go * rstd
    to_ref[...] = bo_ref[...] - mean_c * go * rstd


def _p4_kernel(x1_ref, x2_ref, wo_ref, so_ref, to_ref, out_ref):
    v = jnp.concatenate([x1_ref[0], x2_ref[0]], axis=1)
    o = _dot(wo_ref[...], v, ((1,), (1,)))
    out_ref[0] = jnp.maximum(o * so_ref[...] + to_ref[...], 0.0)


def kernel(x, W1, g1, b1, W2, g2, b2, Wout, gout, bout):
    B, C, N = x.shape
    C1 = W1.shape[0]
    C2 = W2.shape[0]
    CO = Wout.shape[0]
    CV = Wout.shape[1]
    T = N // TILE
    W1a = W1[:, :C]
    Wz = W1[:, C:] - W1[:, :C]

    y, z = pl.pallas_call(
        _p0_kernel,
        grid=(B,),
        in_specs=[
            pl.BlockSpec((1, C, N), lambda b: (b, 0, 0)),
            pl.BlockSpec((C1, C), lambda b: (0, 0)),
            pl.BlockSpec((C1, C), lambda b: (0, 0)),
        ],
        out_specs=[
            pl.BlockSpec((1, N, C1), lambda b: (b, 0, 0)),
            pl.BlockSpec((1, N, C1), lambda b: (b, 0, 0)),
        ],
        out_shape=[
            jax.ShapeDtypeStruct((B, N, C1), F32),
            jax.ShapeDtypeStruct((B, N, C1), F32),
        ],
    )(x, W1a, Wz)

    g, idx, sumh, sumsq = pl.pallas_call(
        _p1_kernel,
        grid=(B, T),
        in_specs=[
            pl.BlockSpec((1, C, TILE), lambda b, t: (b, 0, t)),
            pl.BlockSpec((1, C, N), lambda b, t: (b, 0, 0)),
            pl.BlockSpec((1, N, C1), lambda b, t: (b, 0, 0)),
            pl.BlockSpec((1, TILE, C1), lambda b, t: (b, t, 0)),
        ],
        out_specs=[
            pl.BlockSpec((1, TILE, K, C1), lambda b, t: (b, t, 0, 0)),
            pl.BlockSpec((1, TILE, K), lambda b, t: (b, t, 0)),
            pl.BlockSpec((1, C1), lambda b, t: (0, 0)),
            pl.BlockSpec((1, C1), lambda b, t: (0, 0)),
        ],
        out_shape=[
            jax.ShapeDtypeStruct((B, N, K, C1), F32),
            jax.ShapeDtypeStruct((B, N, K), jnp.int32),
            jax.ShapeDtypeStruct((1, C1), F32),
            jax.ShapeDtypeStruct((1, C1), F32),
        ],
    )(x, x, y, z)
    del idx

    m_edges = B * N * K
    mean1 = sumh / m_edges
    var1 = sumsq / m_edges - mean1 * mean1
    rstd1 = 1.0 / jnp.sqrt(var1 + 1e-5)
    s1 = g1.reshape(1, C1) * rstd1
    t1 = b1.reshape(1, C1) - mean1 * s1

    x1, muhr, mhr = pl.pallas_call(
        _p2_kernel,
        grid=(B, T),
        in_specs=[
            pl.BlockSpec((1, TILE, K, C1), lambda b, t: (b, t, 0, 0)),
            pl.BlockSpec((1, TILE, C1), lambda b, t: (b, t, 0)),
            pl.BlockSpec((1, C1), lambda b, t: (0, 0)),
            pl.BlockSpec((1, C1), lambda b, t: (0, 0)),
        ],
        out_specs=[
            pl.BlockSpec((1, TILE, C1), lambda b, t: (b, t, 0)),
            pl.BlockSpec((1, C1), lambda b, t: (0, 0)),
            pl.BlockSpec((C1, C1), lambda b, t: (0, 0)),
        ],
        out_shape=[
            jax.ShapeDtypeStruct((B, N, C1), F32),
            jax.ShapeDtypeStruct((1, C1), F32),
            jax.ShapeDtypeStruct((C1, C1), F32),
        ],
    )(g, z, s1, t1)

    s2, t2 = pl.pallas_call(
        lambda mu, mm, w2, g2r, b2r, s2o, t2o: _p2b_kernel(
            mu, mm, w2, g2r, b2r, m_edges, s2o, t2o),
        out_shape=[
            jax.ShapeDtypeStruct((1, C2), F32),
            jax.ShapeDtypeStruct((1, C2), F32),
        ],
    )(muhr, mhr, W2, g2.reshape(C2, 1), b2.reshape(C2, 1))

    x2, muv, mv = pl.pallas_call(
        _p3_kernel,
        grid=(B, T),
        in_specs=[
            pl.BlockSpec((1, TILE, K, C1), lambda b, t: (b, t, 0, 0)),
            pl.BlockSpec((1, TILE, C1), lambda b, t: (b, t, 0)),
            pl.BlockSpec((1, C1), lambda b, t: (0, 0)),
            pl.BlockSpec((1, C1), lambda b, t: (0, 0)),
            pl.BlockSpec((1, C2), lambda b, t: (0, 0)),
            pl.BlockSpec((1, C2), lambda b, t: (0, 0)),
            pl.BlockSpec((C2, C1), lambda b, t: (0, 0)),
            pl.BlockSpec((1, TILE, C1), lambda b, t: (b, t, 0)),
        ],
        out_specs=[
            pl.BlockSpec((1, TILE, C2), lambda b, t: (b, t, 0)),
            pl.BlockSpec((1, CV), lambda b, t: (0, 0)),
            pl.BlockSpec((CV, CV), lambda b, t: (0, 0)),
        ],
        out_shape=[
            jax.ShapeDtypeStruct((B, N, C2), F32),
            jax.ShapeDtypeStruct((1, CV), F32),
            jax.ShapeDtypeStruct((CV, CV), F32),
        ],
    )(g, z, s1, t1, s2, t2, W2, x1)

    so, to = pl.pallas_call(
        lambda mu, mm, wo, gor, bor, soo, too: _p3b_kernel(
            mu, mm, wo, gor, bor, B * N, soo, too),
        out_shape=[
            jax.ShapeDtypeStruct((CO, 1), F32),
            jax.ShapeDtypeStruct((CO, 1), F32),
        ],
    )(muv, mv, Wout, gout.reshape(CO, 1), bout.reshape(CO, 1))

    out = pl.pallas_call(
        _p4_kernel,
        grid=(B, T),
        in_specs=[
            pl.BlockSpec((1, TILE, C1), lambda b, t: (b, t, 0)),
            pl.BlockSpec((1, TILE, C2), lambda b, t: (b, t, 0)),
            pl.BlockSpec((CO, CV), lambda b, t: (0, 0)),
            pl.BlockSpec((CO, 1), lambda b, t: (0, 0)),
            pl.BlockSpec((CO, 1), lambda b, t: (0, 0)),
        ],
        out_specs=pl.BlockSpec((1, CO, TILE), lambda b, t: (b, 0, t)),
        out_shape=jax.ShapeDtypeStruct((B, CO, N), F32),
    )(x1, x2, Wout, so, to)
    return out
